# baseline (speedup 1.0000x reference)
import numpy as np
import jax
import jax.numpy as jnp
from jax.experimental import pallas as pl
from jax.experimental.pallas import tpu as pltpu

NB = 512


def _build_w1t(c1_w):
    d = np.zeros((5, 32, 32), np.float32)
    for kj in range(5):
        for jp in range(14):
            for dj in range(2):
                d[kj, 2 * jp + dj + kj, dj * 16 + jp] = 1.0
    b = c1_w[:75].reshape(5, 5, 3, 8)
    w1t = sum(b[:, kj][:, :, None, None, :] * d[kj][None, None, :, :, None]
              for kj in range(5))
    w1t = jnp.pad(w1t.reshape(5, 96, 256), ((0, 0), (0, 32), (0, 0)))
    return w1t.reshape(640, 256)


def _build_w2t(c2_w):
    d = np.zeros((5, 16, 16), np.float32)
    for kj in range(5):
        for j2 in range(5):
            for dj in range(2):
                d[kj, 2 * j2 + dj + kj, dj * 8 + j2] = 1.0
    b = c2_w.reshape(5, 5, 8, 16)
    w2t = sum(b[:, kj][:, None, :, None, :] * d[kj][None, :, None, :, None]
              for kj in range(5))
    return w2t.reshape(640, 256)


def _copies(x_hbm, abuf, sem, n0, slot):
    for r in range(24):
        yield pltpu.make_async_copy(
            x_hbm.at[pl.ds(n0, NB), r, :],
            abuf.at[slot, r],
            sem.at[slot])


def _lenet_kernel(x_hbm, w_ref, b_ref, o_ref, abuf, sem):
    i = pl.program_id(0)
    nsteps = pl.num_programs(0)
    slot = jax.lax.rem(i, 2)
    nslot = jax.lax.rem(i + 1, 2)

    @pl.when(i == 0)
    def _():
        for cp in _copies(x_hbm, abuf, sem, 0, slot):
            cp.start()

    @pl.when(i + 1 < nsteps)
    def _():
        for cp in _copies(x_hbm, abuf, sem, (i + 1) * NB, nslot):
            cp.start()

    for cp in _copies(x_hbm, abuf, sem, i * NB, slot):
        cp.wait()

    abt = abuf[slot].reshape(24 * NB, 128)
    zcol = jnp.zeros((NB, 32), jnp.float32)
    blocks = []
    for h in range(32):
        h4, hm = h // 4, h % 4
        pieces = [abt[(c * 8 + h4) * NB:(c * 8 + h4 + 1) * NB,
                      hm * 32:hm * 32 + 32] for c in range(3)]
        blocks.append(jnp.concatenate(pieces + [zcol], axis=1))
    a = jnp.concatenate(blocks, axis=0).astype(jnp.bfloat16)

    def tap_pair(src, span, k0, w_ref, kw):
        lhs = jnp.concatenate(
            [src[k0 * NB:(span + k0) * NB],
             src[(k0 + 1) * NB:(span + k0 + 1) * NB]], axis=1)
        return jnp.dot(lhs, w_ref[kw * k0:kw * (k0 + 2)],
                       preferred_element_type=jnp.float32)

    acc = tap_pair(a, 28, 0, w_ref, 128)
    acc = acc + tap_pair(a, 28, 2, w_ref, 128)
    acc = acc + jnp.dot(a[4 * NB:32 * NB], w_ref[512:640],
                        preferred_element_type=jnp.float32)
    m = jnp.maximum(acc[:, :128], acc[:, 128:])
    m = m.reshape(14, 2, NB, 128)
    p = jnp.maximum(m[:, 0], m[:, 1]).reshape(14 * NB, 128)
    p = jnp.maximum(p + b_ref[0:1], 0.0).astype(jnp.bfloat16)

    def tap_pair2(src, span, k0):
        lhs = jnp.concatenate(
            [src[k0 * NB:(span + k0) * NB],
             src[(k0 + 1) * NB:(span + k0 + 1) * NB]], axis=1)
        return jnp.dot(lhs, w_ref[640 + 128 * k0:640 + 128 * (k0 + 2)],
                       preferred_element_type=jnp.float32)

    acc2 = tap_pair2(p, 10, 0)
    acc2 = acc2 + tap_pair2(p, 10, 2)
    acc2 = acc2 + jnp.dot(p[4 * NB:14 * NB], w_ref[1152:1280],
                          preferred_element_type=jnp.float32)
    m2 = jnp.maximum(acc2[:, :128], acc2[:, 128:])
    m2 = m2.reshape(5, 2, NB, 128)
    p2 = jnp.maximum(m2[:, 0], m2[:, 1]).reshape(5 * NB, 128)
    p2 = jnp.maximum(p2 + b_ref[1:2], 0.0).astype(jnp.bfloat16)

    nh = NB // 2

    def fc_chain(h0):
        def ftap(k0):
            lhs = jnp.concatenate(
                [p2[k0 * NB + h0:k0 * NB + h0 + nh],
                 p2[(k0 + 1) * NB + h0:(k0 + 1) * NB + h0 + nh]], axis=1)
            return jnp.dot(lhs, w_ref[1280 + 128 * k0:1280 + 128 * (k0 + 2)],
                           preferred_element_type=jnp.float32)
        h = ftap(0) + ftap(2)
        h = h + jnp.dot(p2[4 * NB + h0:4 * NB + h0 + nh], w_ref[1792:1920],
                        preferred_element_type=jnp.float32)
        h = jnp.maximum(h[:, :128] + b_ref[2:3], 0.0).astype(jnp.bfloat16)
        h2 = jnp.dot(h, w_ref[1920:2048], preferred_element_type=jnp.float32)
        h2 = jnp.maximum(h2[:, :128] + b_ref[3:4], 0.0).astype(jnp.bfloat16)
        return (jnp.dot(h2, w_ref[2048:2176],
                        preferred_element_type=jnp.float32)[:, :128]
                + b_ref[4:5])
    o_ref[0:nh] = fc_chain(0)
    o_ref[nh:NB] = fc_chain(nh)


def kernel(x_nchw, c1_w, c1_b, c2_w, c2_b,
           fc1_w, fc1_b, fc2_w, fc2_b, fc3_w, fc3_b):
    n = x_nchw.shape[0]
    wf1 = jnp.pad(fc1_w.reshape(5, 80, 128), ((0, 0), (0, 48), (0, 128)))
    w_all = jnp.concatenate([
        _build_w1t(c1_w),
        _build_w2t(c2_w),
        wf1.reshape(640, 256),
        jnp.pad(fc2_w, ((0, 0), (0, 128))),
        jnp.pad(fc3_w, ((0, 0), (0, 128))),
    ], axis=0).astype(jnp.bfloat16)
    b_all = jnp.concatenate([
        jnp.tile(c1_b, (1, 16)),
        jnp.tile(c2_b, (1, 8)),
        fc1_b, fc2_b, fc3_b,
        jnp.zeros((3, 128), jnp.float32),
    ], axis=0)

    grid = (n // NB,)
    full = lambda shape: pl.BlockSpec(shape, lambda i: (0,) * len(shape))
    out = pl.pallas_call(
        _lenet_kernel,
        out_shape=jax.ShapeDtypeStruct((n, 128), jnp.float32),
        grid_spec=pltpu.PrefetchScalarGridSpec(
            num_scalar_prefetch=0,
            grid=grid,
            in_specs=[
                pl.BlockSpec(memory_space=pl.ANY),
                full((2176, 256)),
                full((8, 128)),
            ],
            out_specs=pl.BlockSpec((NB, 128), lambda i: (i, 0)),
            scratch_shapes=[
                pltpu.VMEM((2, 24, NB, 128), jnp.float32),
                pltpu.SemaphoreType.DMA((2,)),
            ],
        ),
        compiler_params=pltpu.CompilerParams(
            dimension_semantics=("arbitrary",)),
    )(x_nchw.reshape(n, 24, 128), w_all, b_all)
    return out[:, :10]


# speedup vs baseline: 1.0880x; 1.0880x over previous
import numpy as np
import jax
import jax.numpy as jnp
from jax.experimental import pallas as pl
from jax.experimental.pallas import tpu as pltpu

NB = 512


def _build_w1t(c1_w):
    d = np.zeros((5, 32, 32), np.float32)
    for kj in range(5):
        for jp in range(14):
            for dj in range(2):
                d[kj, 2 * jp + dj + kj, dj * 16 + jp] = 1.0
    b = c1_w[:75].reshape(5, 5, 3, 8)
    w1t = jnp.einsum("ijco,jws->icwso", b, d)
    w1t = jnp.pad(w1t.reshape(5, 96, 256), ((0, 0), (0, 32), (0, 0)))
    return w1t.reshape(640, 256).astype(jnp.bfloat16)


def _build_w2t(c2_w):
    d = np.zeros((5, 16, 16), np.float32)
    for kj in range(5):
        for j2 in range(5):
            for dj in range(2):
                d[kj, 2 * j2 + dj + kj, dj * 8 + j2] = 1.0
    b = c2_w.reshape(5, 5, 8, 16)
    w2t = jnp.einsum("ijab,jps->ipasb", b, d)
    return w2t.reshape(640, 256).astype(jnp.bfloat16)


def _copies(x_hbm, abuf, sem, n0, slot):
    for r in range(24):
        yield pltpu.make_async_copy(
            x_hbm.at[pl.ds(n0, NB), r, :],
            abuf.at[slot, r],
            sem.at[slot])


def _lenet_kernel(x_hbm, w1t_ref, b1_ref, w2t_ref, b2_ref,
                  wf1_ref, bf1_ref, wf2_ref, bf2_ref, wf3_ref, bf3_ref,
                  o_ref, abuf, sem):
    i = pl.program_id(0)
    nsteps = pl.num_programs(0)
    slot = jax.lax.rem(i, 2)
    nslot = jax.lax.rem(i + 1, 2)

    @pl.when(i == 0)
    def _():
        for cp in _copies(x_hbm, abuf, sem, 0, slot):
            cp.start()

    @pl.when(i + 1 < nsteps)
    def _():
        for cp in _copies(x_hbm, abuf, sem, (i + 1) * NB, nslot):
            cp.start()

    for cp in _copies(x_hbm, abuf, sem, i * NB, slot):
        cp.wait()

    abt = abuf[slot].reshape(24 * NB, 128)
    zcol = jnp.zeros((NB, 32), jnp.float32)
    blocks = []
    for h in range(32):
        h4, hm = h // 4, h % 4
        pieces = [abt[(c * 8 + h4) * NB:(c * 8 + h4 + 1) * NB,
                      hm * 32:hm * 32 + 32] for c in range(3)]
        blocks.append(jnp.concatenate(pieces + [zcol], axis=1))
    a = jnp.concatenate(blocks, axis=0).astype(jnp.bfloat16)

    def tap_pair(src, span, k0, w_ref, kw):
        lhs = jnp.concatenate(
            [src[k0 * NB:(span + k0) * NB],
             src[(k0 + 1) * NB:(span + k0 + 1) * NB]], axis=1)
        return jnp.dot(lhs, w_ref[kw * k0:kw * (k0 + 2)],
                       preferred_element_type=jnp.float32)

    acc = tap_pair(a, 28, 0, w1t_ref, 128)
    acc = acc + tap_pair(a, 28, 2, w1t_ref, 128)
    acc = acc + jnp.dot(a[4 * NB:32 * NB], w1t_ref[512:640],
                        preferred_element_type=jnp.float32)
    m = jnp.maximum(acc[:, :128], acc[:, 128:])
    m = m.reshape(14, 2, NB, 128)
    p = jnp.maximum(m[:, 0], m[:, 1]).reshape(14 * NB, 128)
    p = jnp.maximum(p + b1_ref[...], 0.0).astype(jnp.bfloat16)

    acc2 = tap_pair(p, 10, 0, w2t_ref, 128)
    acc2 = acc2 + tap_pair(p, 10, 2, w2t_ref, 128)
    acc2 = acc2 + jnp.dot(p[4 * NB:14 * NB], w2t_ref[512:640],
                          preferred_element_type=jnp.float32)
    m2 = jnp.maximum(acc2[:, :128], acc2[:, 128:])
    m2 = m2.reshape(5, 2, NB, 128)
    p2 = jnp.maximum(m2[:, 0], m2[:, 1]).reshape(5 * NB, 128)
    p2 = jnp.maximum(p2 + b2_ref[...], 0.0).astype(jnp.bfloat16)

    nh = NB // 2

    def fc_chain(h0):
        def ftap(k0):
            lhs = jnp.concatenate(
                [p2[k0 * NB + h0:k0 * NB + h0 + nh],
                 p2[(k0 + 1) * NB + h0:(k0 + 1) * NB + h0 + nh]], axis=1)
            return jnp.dot(lhs, wf1_ref[128 * k0:128 * (k0 + 2)],
                           preferred_element_type=jnp.float32)
        h = ftap(0) + ftap(2)
        h = h + jnp.dot(p2[4 * NB + h0:4 * NB + h0 + nh], wf1_ref[512:640],
                        preferred_element_type=jnp.float32)
        h = jnp.maximum(h + bf1_ref[...], 0.0).astype(jnp.bfloat16)
        h2 = jnp.dot(h, wf2_ref[...], preferred_element_type=jnp.float32)
        h2 = jnp.maximum(h2 + bf2_ref[...], 0.0).astype(jnp.bfloat16)
        return (jnp.dot(h2, wf3_ref[...], preferred_element_type=jnp.float32)
                + bf3_ref[...])
    o_ref[0:nh] = fc_chain(0)
    o_ref[nh:NB] = fc_chain(nh)


def kernel(x_nchw, c1_w, c1_b, c2_w, c2_b,
           fc1_w, fc1_b, fc2_w, fc2_b, fc3_w, fc3_b):
    n = x_nchw.shape[0]
    w1t = _build_w1t(c1_w)
    w2t = _build_w2t(c2_w)
    wf1 = jnp.pad(fc1_w.reshape(5, 80, 128), ((0, 0), (0, 48), (0, 0)))
    wf1 = wf1.reshape(640, 128).astype(jnp.bfloat16)
    b1r = jnp.tile(c1_b, (1, 16))
    b2r = jnp.tile(c2_b, (1, 8))

    grid = (n // NB,)
    full = lambda shape: pl.BlockSpec(shape, lambda i: (0,) * len(shape))
    out = pl.pallas_call(
        _lenet_kernel,
        out_shape=jax.ShapeDtypeStruct((n, 128), jnp.float32),
        grid_spec=pltpu.PrefetchScalarGridSpec(
            num_scalar_prefetch=0,
            grid=grid,
            in_specs=[
                pl.BlockSpec(memory_space=pl.ANY),
                full((640, 256)),
                full((1, 128)),
                full((640, 256)),
                full((1, 128)),
                full((640, 128)),
                full((1, 128)),
                full((128, 128)),
                full((1, 128)),
                full((128, 128)),
                full((1, 128)),
            ],
            out_specs=pl.BlockSpec((NB, 128), lambda i: (i, 0)),
            scratch_shapes=[
                pltpu.VMEM((2, 24, NB, 128), jnp.float32),
                pltpu.SemaphoreType.DMA((2,)),
            ],
        ),
        compiler_params=pltpu.CompilerParams(
            dimension_semantics=("arbitrary",)),
    )(x_nchw.reshape(n, 24, 128), w1t, b1r, w2t, b2r,
      wf1, fc1_b, fc2_w.astype(jnp.bfloat16), fc2_b,
      fc3_w.astype(jnp.bfloat16), fc3_b)
    return out[:, :10]
